# Initial kernel scaffold; baseline (speedup 1.0000x reference)
import jax
import jax.numpy as jnp
from jax.experimental import pallas as pl


def kernel(z):
    raise NotImplementedError("write your pallas kernel here")


# speedup vs baseline: 2.2182x; 2.2182x over previous
import jax
import jax.numpy as jnp
from jax.experimental import pallas as pl

STRENGTH = 0.9
THRESHOLD = 0.5

ROWS, COLS = 16384, 1024
BLOCK_ROWS = 1024


def _body(z_ref, o_ref):
    zb = z_ref[...]
    col0 = zb[:, 0:1]
    wet = jax.nn.sigmoid((col0 - THRESHOLD) * 10.0) * STRENGTH
    lane = jax.lax.broadcasted_iota(jnp.int32, zb.shape, 1)
    o_ref[...] = jnp.where(lane == 1, wet, zb)


def kernel(z):
    grid = (ROWS // BLOCK_ROWS,)
    return pl.pallas_call(
        _body,
        grid=grid,
        in_specs=[pl.BlockSpec((BLOCK_ROWS, COLS), lambda i: (i, 0))],
        out_specs=pl.BlockSpec((BLOCK_ROWS, COLS), lambda i: (i, 0)),
        out_shape=jax.ShapeDtypeStruct((ROWS, COLS), jnp.float32),
    )(z)
